# baseline (speedup 1.0000x reference)
import jax
import jax.numpy as jnp
from jax import lax
from jax.experimental import pallas as pl
from jax.experimental.pallas import tpu as pltpu
from jax.experimental.pallas import tpu_sc as plsc

B, L, H = 1024, 200, 3
D = 32
N = B * L
NC, NS, LANES = 2, 16, 16
NW = NC * NS
ROWS_W = N // NW
CHUNK = 128
NIT = ROWS_W // CHUNK
GSL = 128
NG = CHUNK // GSL
TQ = 250000


def _body(idx0_hbm, idx1_hbm, idx2_hbm, feats_hbm, t0, t1, t2, wb_hbm,
          out_hbm, i0_v, i1_v, i2_v, q0_v, q1_v, q2_v, feats_v, wb_v,
          g0_v, g1_v, g2_v, asm_v, sem):
    wid = lax.axis_index("s") * NC + lax.axis_index("c")

    pltpu.sync_copy(wb_hbm, wb_v)
    w_lo = wb_v[pl.ds(0, LANES)]
    w_hi = wb_v[pl.ds(LANES, LANES)]
    b_lo = wb_v[pl.ds(2 * LANES, LANES)]
    b_hi = wb_v[pl.ds(3 * LANES, LANES)]

    ivs = (i0_v, i1_v, i2_v)
    qvs = (q0_v, q1_v, q2_v)
    gvs = (g0_v, g1_v, g2_v)

    def iteration(it, carry):
        base = wid * ROWS_W + it * CHUNK

        pltpu.sync_copy(feats_hbm.at[pl.ds(base, CHUNK)], feats_v)
        for ih, iv in zip((idx0_hbm, idx1_hbm, idx2_hbm), ivs):
            pltpu.sync_copy(ih.at[pl.ds(base, CHUNK)], iv)

        def quot(j, c):
            for l in range(H):
                qvs[l][pl.ds(j * LANES, LANES)] = jax.lax.shift_right_logical(
                    ivs[l][pl.ds(j * LANES, LANES)], 2)
            return c
        lax.fori_loop(0, CHUNK // LANES, quot, 0)

        copies = []
        for l, tbl in enumerate((t0, t1, t2)):
            for j in range(NG):
                copies.append(pltpu.make_async_copy(
                    tbl.at[qvs[l].at[pl.ds(j * GSL, GSL)]],
                    gvs[l].at[pl.ds(j * GSL, GSL), :],
                    sem))
        for c in copies:
            c.start()

        def enc(i16, c):
            fvec = feats_v[pl.ds(i16 * LANES, LANES)]
            for k in range(LANES):
                fv = jnp.full((LANES,), fvec[k])
                asm_v[i16 * LANES + k, pl.ds(3 * D, LANES)] = fv * w_lo + b_lo
                asm_v[i16 * LANES + k, pl.ds(3 * D + LANES, LANES)] = (
                    fv * w_hi + b_hi)
            return c
        lax.fori_loop(0, CHUNK // LANES, enc, 0)

        for c in copies:
            c.wait()

        def extract(i16, c):
            for l in range(H):
                pvec = jax.lax.shift_left(
                    jnp.bitwise_and(ivs[l][pl.ds(i16 * LANES, LANES)], 3), 5)
                for k in range(LANES):
                    i = i16 * LANES + k
                    off = pvec[k]
                    asm_v[i, pl.ds(l * D, LANES)] = (
                        gvs[l][i, pl.ds(off, LANES)])
                    asm_v[i, pl.ds(l * D + LANES, LANES)] = (
                        gvs[l][i, pl.ds(off + LANES, LANES)])
            return c
        lax.fori_loop(0, CHUNK // LANES, extract, 0)

        pltpu.sync_copy(asm_v, out_hbm.at[pl.ds(base, CHUNK), :])
        return carry

    lax.fori_loop(0, NIT, iteration, 0)


@jax.jit
def _sc_embed(idx0, idx1, idx2, feats, t0, t1, t2, wb):
    mesh = plsc.VectorSubcoreMesh(core_axis_name="c", subcore_axis_name="s",
                                  num_cores=NC, num_subcores=NS)
    f = pl.kernel(
        _body,
        out_type=jax.ShapeDtypeStruct((N, (H + 1) * D), jnp.float32),
        mesh=mesh,
        compiler_params=pltpu.CompilerParams(use_tc_tiling_on_sc=True),
        scratch_types=[
            pltpu.VMEM((CHUNK,), jnp.int32),
            pltpu.VMEM((CHUNK,), jnp.int32),
            pltpu.VMEM((CHUNK,), jnp.int32),
            pltpu.VMEM((CHUNK,), jnp.int32),
            pltpu.VMEM((CHUNK,), jnp.int32),
            pltpu.VMEM((CHUNK,), jnp.int32),
            pltpu.VMEM((CHUNK,), jnp.float32),
            pltpu.VMEM((4 * LANES,), jnp.float32),
            pltpu.VMEM((CHUNK, 4 * D), jnp.float32),
            pltpu.VMEM((CHUNK, 4 * D), jnp.float32),
            pltpu.VMEM((CHUNK, 4 * D), jnp.float32),
            pltpu.VMEM((CHUNK, (H + 1) * D), jnp.float32),
            pltpu.SemaphoreType.DMA,
        ],
    )
    return f(idx0, idx1, idx2, feats, t0, t1, t2, wb)


def kernel(tokens, features, T0, T1, T2, W, b):
    tok = tokens.reshape(N, H)
    feats = features.reshape(N)
    wb = jnp.concatenate([W.reshape(D), b.reshape(D)])
    out = _sc_embed(tok[:, 0], tok[:, 1], tok[:, 2], feats,
                    T0.reshape(TQ, 4 * D), T1.reshape(TQ, 4 * D),
                    T2.reshape(TQ, 4 * D), wb)
    return out.reshape(B, L, (H + 1) * D)


# speedup vs baseline: 1.2436x; 1.2436x over previous
import jax
import jax.numpy as jnp
from jax import lax
from jax.experimental import pallas as pl
from jax.experimental.pallas import tpu as pltpu
from jax.experimental.pallas import tpu_sc as plsc

B, L, H = 1024, 200, 3
D = 32
N = B * L
NC, NS, LANES = 2, 16, 16
NW = NC * NS
ROWS_W = N // NW
CHUNK = 640
NIT = ROWS_W // CHUNK
GSL = 128
NG = CHUNK // GSL


def _body(idx0_hbm, idx1_hbm, idx2_hbm, feats_hbm, t0, t1, t2, wb_hbm,
          out_hbm, idx0_v, idx1_v, idx2_v, feats_v, wb_v,
          g0_v, g1_v, g2_v, e_v, sem):
    wid = lax.axis_index("s") * NC + lax.axis_index("c")

    pltpu.sync_copy(wb_hbm, wb_v)
    w_lo = wb_v[pl.ds(0, LANES)]
    w_hi = wb_v[pl.ds(LANES, LANES)]
    b_lo = wb_v[pl.ds(2 * LANES, LANES)]
    b_hi = wb_v[pl.ds(3 * LANES, LANES)]

    def iteration(it, carry):
        base = wid * ROWS_W + it * CHUNK

        pltpu.sync_copy(feats_hbm.at[pl.ds(base, CHUNK)], feats_v)
        pltpu.sync_copy(idx0_hbm.at[pl.ds(base, CHUNK)], idx0_v)
        pltpu.sync_copy(idx1_hbm.at[pl.ds(base, CHUNK)], idx1_v)
        pltpu.sync_copy(idx2_hbm.at[pl.ds(base, CHUNK)], idx2_v)

        copies = []
        for tbl, iv, gv in ((t0, idx0_v, g0_v), (t1, idx1_v, g1_v),
                            (t2, idx2_v, g2_v)):
            for j in range(NG):
                copies.append(pltpu.make_async_copy(
                    tbl.at[iv.at[pl.ds(j * GSL, GSL)]],
                    gv.at[pl.ds(j * GSL, GSL), :],
                    sem))
        for c in copies:
            c.start()

        def enc(i16, c):
            fvec = feats_v[pl.ds(i16 * LANES, LANES)]
            for k in range(LANES):
                fv = jnp.full((LANES,), fvec[k])
                e_v[i16 * LANES + k, pl.ds(0, LANES)] = fv * w_lo + b_lo
                e_v[i16 * LANES + k, pl.ds(LANES, LANES)] = fv * w_hi + b_hi
            return c
        lax.fori_loop(0, CHUNK // LANES, enc, 0)

        for c in copies:
            c.wait()

        for s, gv in enumerate((g0_v, g1_v, g2_v, e_v)):
            pltpu.sync_copy(gv, out_hbm.at[pl.ds(base, CHUNK), s, :])
        return carry

    lax.fori_loop(0, NIT, iteration, 0)


@jax.jit
def _sc_embed(idx0, idx1, idx2, feats, t0, t1, t2, wb):
    mesh = plsc.VectorSubcoreMesh(core_axis_name="c", subcore_axis_name="s",
                                  num_cores=NC, num_subcores=NS)
    f = pl.kernel(
        _body,
        out_type=jax.ShapeDtypeStruct((N, H + 1, D), jnp.float32),
        mesh=mesh,
        compiler_params=pltpu.CompilerParams(use_tc_tiling_on_sc=False),
        scratch_types=[
            pltpu.VMEM((CHUNK,), jnp.int32),
            pltpu.VMEM((CHUNK,), jnp.int32),
            pltpu.VMEM((CHUNK,), jnp.int32),
            pltpu.VMEM((CHUNK,), jnp.float32),
            pltpu.VMEM((4 * LANES,), jnp.float32),
            pltpu.VMEM((CHUNK, D), jnp.float32),
            pltpu.VMEM((CHUNK, D), jnp.float32),
            pltpu.VMEM((CHUNK, D), jnp.float32),
            pltpu.VMEM((CHUNK, D), jnp.float32),
            pltpu.SemaphoreType.DMA,
        ],
    )
    return f(idx0, idx1, idx2, feats, t0, t1, t2, wb)


def kernel(tokens, features, T0, T1, T2, W, b):
    tok = tokens.reshape(N, H)
    feats = features.reshape(N)
    wb = jnp.concatenate([W.reshape(D), b.reshape(D)])
    out = _sc_embed(tok[:, 0], tok[:, 1], tok[:, 2], feats, T0, T1, T2, wb)
    return out.reshape(B, L, (H + 1) * D)


# speedup vs baseline: 1.4083x; 1.1324x over previous
import jax
import jax.numpy as jnp
from jax import lax
from jax.experimental import pallas as pl
from jax.experimental.pallas import tpu as pltpu
from jax.experimental.pallas import tpu_sc as plsc

B, L, H = 1024, 200, 3
D = 32
N = B * L
NC, NS, LANES = 2, 16, 16
NW = NC * NS
ROWS_W = N // NW
CHUNK = 128
NIT = ROWS_W // CHUNK
V = 1000000

TCB = 2048
TGRID = (V + TCB - 1) // TCB
YR = TGRID * (TCB // 4)


def _tc_body(x0, x1, x2, y0, y1, y2):
    for x, y in ((x0, y0), (x1, y1), (x2, y2)):
        xv = x[...]
        for a in range(TCB // 128):
            q, p = a // 4, a % 4
            xa = xv[:, 128 * a:128 * (a + 1)]
            y[128 * q:128 * (q + 1), 32 * p:32 * (p + 1)] = xa.T


def _tc_repack(t0t, t1t, t2t):
    in_spec = pl.BlockSpec((32, TCB), lambda j: (0, j))
    out_spec = pl.BlockSpec((TCB // 4, 128), lambda j: (j, 0))
    return pl.pallas_call(
        _tc_body,
        grid=(TGRID,),
        in_specs=[in_spec] * H,
        out_specs=[out_spec] * H,
        out_shape=[jax.ShapeDtypeStruct((YR, 128), jnp.float32)] * H,
    )(t0t, t1t, t2t)


def _gbody(idx0_hbm, idx1_hbm, idx2_hbm, feats_hbm, y0, y1, y2, wb_hbm,
           out_hbm, iv0, iv1, iv2, rv0, rv1, rv2, ov0, ov1, ov2,
           feats_v, wb_v, g0_v, g1_v, g2_v, asm_v, sem):
    wid = lax.axis_index("s") * NC + lax.axis_index("c")

    pltpu.sync_copy(wb_hbm, wb_v)
    w_lo = wb_v[pl.ds(0, LANES)]
    w_hi = wb_v[pl.ds(LANES, LANES)]
    b_lo = wb_v[pl.ds(2 * LANES, LANES)]
    b_hi = wb_v[pl.ds(3 * LANES, LANES)]

    def iteration(it, carry):
        base = wid * ROWS_W + it * CHUNK

        pltpu.sync_copy(feats_hbm.at[pl.ds(base, CHUNK)], feats_v)
        pltpu.sync_copy(idx0_hbm.at[pl.ds(base, CHUNK)], iv0)
        pltpu.sync_copy(idx1_hbm.at[pl.ds(base, CHUNK)], iv1)
        pltpu.sync_copy(idx2_hbm.at[pl.ds(base, CHUNK)], iv2)

        def rowcalc(j, c):
            for iv, rv, ov in ((iv0, rv0, ov0), (iv1, rv1, ov1),
                               (iv2, rv2, ov2)):
                v = iv[pl.ds(j * LANES, LANES)]
                rv[pl.ds(j * LANES, LANES)] = (
                    (v >> 9) << 7) | (v & 127)
                ov[pl.ds(j * LANES, LANES)] = ((v >> 7) & 3) << 5
            return c
        lax.fori_loop(0, CHUNK // LANES, rowcalc, 0)

        copies = []
        for tbl, rv, gv in ((y0, rv0, g0_v), (y1, rv1, g1_v),
                            (y2, rv2, g2_v)):
            copies.append(pltpu.make_async_copy(
                tbl.at[rv.at[pl.ds(0, CHUNK)]], gv, sem))
        for c in copies:
            c.start()

        def enc(i16, c):
            fvec = feats_v[pl.ds(i16 * LANES, LANES)]
            for k in range(LANES):
                fv = jnp.full((LANES,), fvec[k])
                asm_v[i16 * LANES + k, pl.ds(96, LANES)] = fv * w_lo + b_lo
                asm_v[i16 * LANES + k, pl.ds(112, LANES)] = fv * w_hi + b_hi
            return c
        lax.fori_loop(0, CHUNK // LANES, enc, 0)

        for c in copies:
            c.wait()

        def extract(i16, c):
            o0 = ov0[pl.ds(i16 * LANES, LANES)]
            o1 = ov1[pl.ds(i16 * LANES, LANES)]
            o2 = ov2[pl.ds(i16 * LANES, LANES)]
            for k in range(LANES):
                r = i16 * LANES + k
                for gv, ov, lane0 in ((g0_v, o0, 0), (g1_v, o1, 32),
                                      (g2_v, o2, 64)):
                    off = ov[k]
                    asm_v[r, pl.ds(lane0, LANES)] = gv[r, pl.ds(off, LANES)]
                    asm_v[r, pl.ds(lane0 + LANES, LANES)] = (
                        gv[r, pl.ds(off + LANES, LANES)])
            return c
        lax.fori_loop(0, CHUNK // LANES, extract, 0)

        pltpu.sync_copy(asm_v, out_hbm.at[pl.ds(base, CHUNK), :])
        return carry

    lax.fori_loop(0, NIT, iteration, 0)


def _sc_embed(idx0, idx1, idx2, feats, y0, y1, y2, wb):
    mesh = plsc.VectorSubcoreMesh(core_axis_name="c", subcore_axis_name="s",
                                  num_cores=NC, num_subcores=NS)
    f = pl.kernel(
        _gbody,
        out_type=jax.ShapeDtypeStruct((N, 4 * D), jnp.float32),
        mesh=mesh,
        compiler_params=pltpu.CompilerParams(use_tc_tiling_on_sc=True),
        scratch_types=[
            pltpu.VMEM((CHUNK,), jnp.int32),
            pltpu.VMEM((CHUNK,), jnp.int32),
            pltpu.VMEM((CHUNK,), jnp.int32),
            pltpu.VMEM((CHUNK,), jnp.int32),
            pltpu.VMEM((CHUNK,), jnp.int32),
            pltpu.VMEM((CHUNK,), jnp.int32),
            pltpu.VMEM((CHUNK,), jnp.int32),
            pltpu.VMEM((CHUNK,), jnp.int32),
            pltpu.VMEM((CHUNK,), jnp.int32),
            pltpu.VMEM((CHUNK,), jnp.float32),
            pltpu.VMEM((4 * LANES,), jnp.float32),
            pltpu.VMEM((CHUNK, 128), jnp.float32),
            pltpu.VMEM((CHUNK, 128), jnp.float32),
            pltpu.VMEM((CHUNK, 128), jnp.float32),
            pltpu.VMEM((CHUNK, 128), jnp.float32),
            pltpu.SemaphoreType.DMA,
        ],
    )
    return f(idx0, idx1, idx2, feats, y0, y1, y2, wb)


@jax.jit
def _run(tokens, features, T0, T1, T2, W, b):
    tok = tokens.reshape(N, H)
    feats = features.reshape(N)
    wb = jnp.concatenate([W.reshape(D), b.reshape(D)])
    y0, y1, y2 = _tc_repack(T0.T, T1.T, T2.T)
    out = _sc_embed(tok[:, 0], tok[:, 1], tok[:, 2], feats, y0, y1, y2, wb)
    return out.reshape(B, L, (H + 1) * D)


def kernel(tokens, features, T0, T1, T2, W, b):
    return _run(tokens, features, T0, T1, T2, W, b)
